# baseline (device time: 254764 ns/iter reference)
import jax
import jax.numpy as jnp
from jax import lax
from jax.experimental import pallas as pl
from jax.experimental.pallas import tpu as pltpu

B, SQ, H, D = 4, 32, 8, 128
SKV_SHARD = 4096
K_CHUNK = 512
N_CHUNKS = SKV_SHARD // K_CHUNK
SCALE = D ** -0.5
HD = H * D

DeviceIdType = getattr(pl, "DeviceIdType", None) or pltpu.DeviceIdType
semaphore_signal = getattr(pl, "semaphore_signal", None) or pltpu.semaphore_signal
semaphore_wait = getattr(pl, "semaphore_wait", None) or pltpu.semaphore_wait


def kernel(Q, K, V):
    def body(q_ref, k_ref, v_ref, out_ref,
             acc_ref, m_ref, l_ref,
             peer_o, peer_m, peer_l,
             send_sems, recv_sems):
        step = pl.program_id(0)
        my_x = lax.axis_index("x")
        my_y = lax.axis_index("y")
        my_z = lax.axis_index("z")
        peer = (1 - my_x, my_y, my_z)

        @pl.when(step == 0)
        def _init():
            barrier_sem = pltpu.get_barrier_semaphore()
            semaphore_signal(barrier_sem, inc=1, device_id=peer,
                             device_id_type=DeviceIdType.MESH)
            semaphore_wait(barrier_sem, 1)
            m_ref[...] = jnp.full((B, H, SQ, 1), -1e30, jnp.float32)
            l_ref[...] = jnp.zeros((B, H, SQ, 1), jnp.float32)
            acc_ref[...] = jnp.zeros((B, SQ, HD), jnp.float32)

        for b in range(B):
            for h in range(H):
                hs = slice(h * D, (h + 1) * D)
                q = q_ref[b, :, hs].astype(jnp.bfloat16)
                k = k_ref[b, :, hs].astype(jnp.bfloat16)
                v = v_ref[b, :, hs].astype(jnp.bfloat16)
                s = lax.dot_general(
                    q, k, (((1,), (1,)), ((), ())),
                    preferred_element_type=jnp.float32,
                ) * SCALE
                m_old = m_ref[b, h]
                m_new = jnp.maximum(m_old, jnp.max(s, axis=1, keepdims=True))
                alpha = jnp.exp(m_old - m_new)
                p = jnp.exp(s - m_new)
                pv = lax.dot_general(
                    p.astype(jnp.bfloat16), v, (((1,), (0,)), ((), ())),
                    preferred_element_type=jnp.float32,
                )
                l_ref[b, h] = l_ref[b, h] * alpha + jnp.sum(p, axis=1, keepdims=True)
                acc_ref[b, :, hs] = acc_ref[b, :, hs] * alpha + pv
                m_ref[b, h] = m_new

        @pl.when(step == N_CHUNKS - 1)
        def _exchange_and_merge():
            rdmas = []
            for i, (src, dst) in enumerate(
                [(acc_ref, peer_o), (m_ref, peer_m), (l_ref, peer_l)]
            ):
                rdma = pltpu.make_async_remote_copy(
                    src_ref=src, dst_ref=dst,
                    send_sem=send_sems.at[i], recv_sem=recv_sems.at[i],
                    device_id=peer, device_id_type=DeviceIdType.MESH,
                )
                rdma.start()
                rdmas.append(rdma)
            for rdma in rdmas:
                rdma.wait()

            for b in range(B):
                for h in range(H):
                    hs = slice(h * D, (h + 1) * D)
                    m1, m2 = m_ref[b, h], peer_m[b, h]
                    l1, l2 = l_ref[b, h], peer_l[b, h]
                    m = jnp.maximum(m1, m2)
                    a1 = jnp.exp(m1 - m)
                    a2 = jnp.exp(m2 - m)
                    l = l1 * a1 + l2 * a2
                    out_ref[b, :, hs] = (
                        acc_ref[b, :, hs] * a1 + peer_o[b, :, hs] * a2
                    ) / l

    grid = (N_CHUNKS,)
    out = pl.pallas_call(
        body,
        grid=grid,
        out_shape=jax.ShapeDtypeStruct((B, SQ, HD), jnp.float32),
        in_specs=[
            pl.BlockSpec((B, SQ, HD), lambda i: (0, 0, 0),
                         memory_space=pltpu.VMEM),
            pl.BlockSpec((B, K_CHUNK, HD), lambda i: (0, i, 0),
                         memory_space=pltpu.VMEM),
            pl.BlockSpec((B, K_CHUNK, HD), lambda i: (0, i, 0),
                         memory_space=pltpu.VMEM),
        ],
        out_specs=pl.BlockSpec((B, SQ, HD), lambda i: (0, 0, 0),
                               memory_space=pltpu.VMEM),
        scratch_shapes=[
            pltpu.VMEM((B, SQ, HD), jnp.float32),
            pltpu.VMEM((B, H, SQ, 1), jnp.float32),
            pltpu.VMEM((B, H, SQ, 1), jnp.float32),
            pltpu.VMEM((B, SQ, HD), jnp.float32),
            pltpu.VMEM((B, H, SQ, 1), jnp.float32),
            pltpu.VMEM((B, H, SQ, 1), jnp.float32),
            pltpu.SemaphoreType.DMA((3,)),
            pltpu.SemaphoreType.DMA((3,)),
        ],
        compiler_params=pltpu.CompilerParams(
            collective_id=0,
            dimension_semantics=("arbitrary",),
            vmem_limit_bytes=100 * 1024 * 1024,
        ),
    )(
        Q.reshape(B, SQ, HD),
        K.reshape(B, SKV_SHARD, HD),
        V.reshape(B, SKV_SHARD, HD),
    )
    return out.reshape(B, SQ, H, D)


# device time: 38592 ns/iter; 6.6015x vs baseline; 6.6015x over previous
import jax
import jax.numpy as jnp
from jax import lax
from jax.experimental import pallas as pl
from jax.experimental.pallas import tpu as pltpu

B, SQ, H, D = 4, 32, 8, 128
SKV = 4096
SCALE = D ** -0.5

DeviceIdType = getattr(pl, "DeviceIdType", None) or pltpu.DeviceIdType
semaphore_signal = getattr(pl, "semaphore_signal", None) or pltpu.semaphore_signal
semaphore_wait = getattr(pl, "semaphore_wait", None) or pltpu.semaphore_wait


def kernel(Q, K, V):
    def body(q_ref, k_ref, v_ref, out_ref,
             kbuf, vbuf,
             obuf, mbuf, lbuf,
             pobuf, pmbuf, plbuf,
             hbuf,
             dma_sems, x_send, x_recv, ag_send, ag_recv):
        my_x = lax.axis_index("x")
        my_y = lax.axis_index("y")
        my_z = lax.axis_index("z")
        g = 4 * my_y + my_z
        s0 = 2 * my_z + my_y
        x_peer = (1 - my_x, my_y, my_z)
        y_peer = (my_x, 1 - my_y, my_z)
        z1_peer = (my_x, my_y, my_z ^ 1)
        z2_peer = (my_x, my_y, my_z ^ 2)
        peers = [x_peer, y_peer, z1_peer, z2_peer]

        barrier_sem = pltpu.get_barrier_semaphore()
        for p in peers:
            semaphore_signal(barrier_sem, inc=1, device_id=p,
                             device_id_type=DeviceIdType.MESH)
        semaphore_wait(barrier_sem, len(peers))

        cp_k = pltpu.make_async_copy(
            k_ref.at[:, :, pl.ds(g, 1), :], kbuf, dma_sems.at[0])
        cp_v = pltpu.make_async_copy(
            v_ref.at[:, :, pl.ds(g, 1), :], vbuf, dma_sems.at[1])
        cp_k.start()
        cp_v.start()
        cp_k.wait()

        for b in range(B):
            q = q_ref[b, :, pl.ds(g, 1), :][:, 0, :]
            k = kbuf[b, :, 0, :]
            s = lax.dot_general(
                q, k, (((1,), (1,)), ((), ())),
                preferred_element_type=jnp.float32,
            ) * SCALE
            m = jnp.max(s, axis=1, keepdims=True)
            p = jnp.exp(s - m)
            if b == 0:
                cp_v.wait()
            pv = lax.dot_general(
                p, vbuf[b, :, 0, :], (((1,), (0,)), ((), ())),
                preferred_element_type=jnp.float32,
            )
            mbuf[b] = m
            lbuf[b] = jnp.sum(p, axis=1, keepdims=True)
            obuf[b] = pv

        rdmas = []
        for i, (src, dst) in enumerate(
            [(obuf, pobuf), (mbuf, pmbuf), (lbuf, plbuf)]
        ):
            rdma = pltpu.make_async_remote_copy(
                src_ref=src, dst_ref=dst,
                send_sem=x_send.at[i], recv_sem=x_recv.at[i],
                device_id=x_peer, device_id_type=DeviceIdType.MESH,
            )
            rdma.start()
            rdmas.append(rdma)
        for rdma in rdmas:
            rdma.wait()

        m1, m2 = mbuf[...], pmbuf[...]
        l1, l2 = lbuf[...], plbuf[...]
        mm = jnp.maximum(m1, m2)
        a1 = jnp.exp(m1 - mm)
        a2 = jnp.exp(m2 - mm)
        ll = l1 * a1 + l2 * a2
        merged = (obuf[...] * a1 + pobuf[...] * a2) / ll
        hbuf[pl.ds(s0, 1)] = merged[None]

        for p, partner in enumerate([y_peer, z1_peer, z2_peer]):
            size = 1 << p
            base = s0 & (~(size - 1) & 7)
            ag = pltpu.make_async_remote_copy(
                src_ref=hbuf.at[pl.ds(base, size)],
                dst_ref=hbuf.at[pl.ds(base, size)],
                send_sem=ag_send.at[p], recv_sem=ag_recv.at[p],
                device_id=partner, device_id_type=DeviceIdType.MESH,
            )
            ag.start()
            ag.wait()

        for s_ in range(H):
            h = 4 * (s_ & 1) + (s_ >> 1)
            for b in range(B):
                out_ref[b, :, h, :] = hbuf[s_, b]

    return pl.pallas_call(
        body,
        out_shape=jax.ShapeDtypeStruct((B, SQ, H, D), jnp.float32),
        in_specs=[
            pl.BlockSpec(memory_space=pltpu.VMEM),
            pl.BlockSpec(memory_space=pl.ANY),
            pl.BlockSpec(memory_space=pl.ANY),
        ],
        out_specs=pl.BlockSpec(memory_space=pltpu.VMEM),
        scratch_shapes=[
            pltpu.VMEM((B, SKV, 1, D), jnp.float32),
            pltpu.VMEM((B, SKV, 1, D), jnp.float32),
            pltpu.VMEM((B, SQ, D), jnp.float32),
            pltpu.VMEM((B, SQ, 1), jnp.float32),
            pltpu.VMEM((B, SQ, 1), jnp.float32),
            pltpu.VMEM((B, SQ, D), jnp.float32),
            pltpu.VMEM((B, SQ, 1), jnp.float32),
            pltpu.VMEM((B, SQ, 1), jnp.float32),
            pltpu.VMEM((H, B, SQ, D), jnp.float32),
            pltpu.SemaphoreType.DMA((2,)),
            pltpu.SemaphoreType.DMA((3,)),
            pltpu.SemaphoreType.DMA((3,)),
            pltpu.SemaphoreType.DMA((3,)),
            pltpu.SemaphoreType.DMA((3,)),
        ],
        compiler_params=pltpu.CompilerParams(
            collective_id=0,
            vmem_limit_bytes=100 * 1024 * 1024,
        ),
    )(Q, K, V)
